# baseline (speedup 1.0000x reference)
import functools

import jax
import jax.numpy as jnp
from jax import lax
from jax.experimental import pallas as pl
from jax.experimental.pallas import tpu as pltpu
from jax.experimental.pallas import tpu_sc as plsc

N = 10000
E = 320000
D = 128
NH = 64
NO = 16
NG = 128
NC = 10
MID = 256
LAST = 32
IN_MLP = 2 * NH + 2 * NO
EPS = 1e-5

NP = 10240
BLK = 128
NTILES = 32
NBLK_PER_TILE = 80
RBPT = 78
LEFT0 = NTILES * RBPT
DE = 8
DR = 4
ROWS_PER_TILE = NP // 16
ZR = 128
NEG = float("-inf")


def _dotT(a, b):
    return lax.dot_general(a, b, (((1,), (1,)), ((), ())),
                           preferred_element_type=jnp.float32)


def _tc1_body(x_ref, w_ref, o_ref):
    o_ref[0:N, :] = _dotT(x_ref[...], w_ref[...])
    o_ref[N:NP, :] = jnp.zeros((NP - N, NH), jnp.float32)


def _tc2_body(s1p_ref, b1_ref, w2_ref, h1_ref, h2_ref):
    s1 = s1p_ref[0:N, :] + s1p_ref[NP:NP + N, :]
    h1 = jnp.maximum(s1 + b1_ref[...], 0.0)
    h1_ref[...] = h1
    h2_ref[0:N, :] = _dotT(h1, w2_ref[...])
    h2_ref[N:NP, :] = jnp.zeros((NP - N, NO), jnp.float32)


def _bn_eval(x, g, b):
    return g * (x * (1.0 / (1.0 + EPS) ** 0.5)) + b


def _segmax(y, bb, C):
    s = 1
    while s < N:
        bs = jnp.concatenate(
            [jnp.full((s, 1), -1, jnp.int32), bb[:N - s]], axis=0)
        same = bb == bs
        ms = jnp.concatenate(
            [jnp.full((s, C), NEG, jnp.float32), y[:N - s]], axis=0)
        y = jnp.maximum(y, jnp.where(same, ms, NEG))
        s *= 2
    return y


def _pool(y, bb):
    C = y.shape[1]
    ymax = _segmax(y, bb, C)
    gidx = lax.broadcasted_iota(jnp.int32, (1, NG), 1)
    monehot = (bb == gidx).astype(jnp.float32)
    gsum = lax.dot_general(monehot, y, (((0,), (0,)), ((), ())),
                           preferred_element_type=jnp.float32)
    bnext = jnp.concatenate([bb[1:], jnp.full((1, 1), -1, jnp.int32)], axis=0)
    mend = monehot * (bnext != bb).astype(jnp.float32)
    gmax = lax.dot_general(mend, ymax, (((0,), (0,)), ((), ())),
                           preferred_element_type=jnp.float32)
    return gmax, gsum


def _tc3_body(h1_ref, s2p_ref, b_ref, b2_ref, bn0g_ref, bn0b_ref,
              fc1w_ref, fc1b_ref, bn1g_ref, bn1b_ref, fc2w_ref, fc2b_ref,
              clsw_ref, clsb_ref, o_ref):
    s2 = s2p_ref[0:N, :] + s2p_ref[NP:NP + N, :] + b2_ref[...]
    bb = b_ref[...]
    C = NH + NO
    y = jnp.concatenate([h1_ref[...], s2, jnp.ones((N, 1), jnp.float32)],
                        axis=1)
    gmax, gsum = _pool(y, bb)

    cnt = jnp.maximum(gsum[:, C:C + 1], 1.0)
    xf = jnp.concatenate([
        gmax[:, :NH], gsum[:, :NH] / cnt,
        gmax[:, NH:C], gsum[:, NH:C] / cnt,
    ], axis=1)

    xf = jnp.maximum(_dotT(_bn_eval(xf, bn0g_ref[...], bn0b_ref[...]),
                           fc1w_ref[...]) + fc1b_ref[...], 0.0)
    xf = jnp.maximum(_dotT(_bn_eval(xf, bn1g_ref[...], bn1b_ref[...]),
                           fc2w_ref[...]) + fc2b_ref[...], 0.0)
    logits = _dotT(xf, clsw_ref[...]) + clsb_ref[...]
    m = jnp.max(logits, axis=1, keepdims=True)
    lse = m + jnp.log(jnp.sum(jnp.exp(logits - m), axis=1, keepdims=True))
    o_ref[...] = logits - lse


def _make_sc_edge(F):
    mesh = plsc.VectorSubcoreMesh(core_axis_name="c", subcore_axis_name="s",
                                  num_cores=2, num_subcores=16)

    @functools.partial(
        pl.kernel,
        out_type=jax.ShapeDtypeStruct((2 * NP, F), jnp.float32),
        mesh=mesh,
        scratch_types=[
            pltpu.VMEM((DE, 2, BLK), jnp.int32),
            pltpu.VMEM((DE, BLK), jnp.float32),
            pltpu.VMEM((DR, BLK, F), jnp.float32),
            pltpu.VMEM((ZR, F), jnp.float32),
            pltpu.VMEM_SHARED((NP, F), jnp.float32),
            pltpu.VMEM_SHARED((NP, F), jnp.float32),
            pltpu.SemaphoreType.DMA,
            pltpu.SemaphoreType.DMA,
            pltpu.SemaphoreType.DMA,
        ],
        compiler_params=pltpu.CompilerParams(use_tc_tiling_on_sc=False),
    )
    def sc_edge(src_hbm, dst_hbm, ew_hbm, h_hbm, out_hbm, ebuf, wbuf, rows,
                zbuf, acc, hs, esem, gsem, ssem):
        c = lax.axis_index("c")
        s = lax.axis_index("s")
        wid = c * 16 + s

        def zb(i, carry):
            for q in range(F // 16):
                zbuf[i, pl.ds(q * 16, 16)] = jnp.zeros((16,), jnp.float32)
            return carry
        lax.fori_loop(0, ZR, zb, 0)
        for k in range(ROWS_PER_TILE // ZR):
            off = s * ROWS_PER_TILE + k * ZR
            pltpu.sync_copy(zbuf, acc.at[pl.ds(off, ZR)])
            pltpu.sync_copy(h_hbm.at[pl.ds(off, ZR)], hs.at[pl.ds(off, ZR)])
        plsc.subcore_barrier()

        blk0 = wid * RBPT

        def eload(j, slot):
            blk = jnp.where(j < RBPT, blk0 + j,
                            jnp.where(j == RBPT, LEFT0 + (wid % 4), 0))
            off = blk * BLK
            pltpu.async_copy(src_hbm.at[pl.ds(off, BLK)], ebuf.at[slot, 0],
                             esem)
            pltpu.async_copy(dst_hbm.at[pl.ds(off, BLK)], ebuf.at[slot, 1],
                             esem)
            pltpu.async_copy(ew_hbm.at[pl.ds(off, BLK)], wbuf.at[slot], esem)

        def eload_wait():
            pltpu.make_async_copy(src_hbm.at[pl.ds(0, BLK)], ebuf.at[0, 0],
                                  esem).wait()
            pltpu.make_async_copy(dst_hbm.at[pl.ds(0, BLK)], ebuf.at[0, 1],
                                  esem).wait()
            pltpu.make_async_copy(ew_hbm.at[pl.ds(0, BLK)], wbuf.at[0],
                                  esem).wait()

        def gissue(eslot, rslot):
            pltpu.async_copy(hs.at[ebuf.at[eslot, 0]], rows.at[rslot], gsem)

        def gwait():
            pltpu.make_async_copy(hs.at[ebuf.at[0, 0]], rows.at[0],
                                  gsem).wait()

        def sissue(eslot, rslot):
            pltpu.async_copy(rows.at[rslot], acc.at[ebuf.at[eslot, 1]], ssem,
                             add=True)

        def swait():
            pltpu.make_async_copy(rows.at[0], acc.at[ebuf.at[0, 1]],
                                  ssem).wait()

        def scale(eslot, rslot, mask=None):
            def sbody(e16, carry):
                ew16 = wbuf[eslot, pl.ds(e16 * 16, 16)]
                ew16 = jnp.where(ew16 != ew16, 0.0, ew16)
                if mask is not None:
                    ew16 = ew16 * mask
                base = e16 * 16
                for e in range(16):
                    sv = ew16[e]
                    for q in range(F // 16):
                        sl = pl.ds(q * 16, 16)
                        rows[rslot, base + e, sl] = rows[rslot, base + e, sl] * sv
                return carry
            lax.fori_loop(0, BLK // 16, sbody, 0)

        def do_slot(j, jm8, drain_scatter, load_ahead, issue_next,
                    mask=None):
            if drain_scatter:
                swait()
            if issue_next:
                eload_wait()
                gissue((jm8 + 2) % DE, (jm8 + 2) % DR)
            gwait()
            scale(jm8, jm8 % DR, mask)
            sissue(jm8, jm8 % DR)
            if load_ahead:
                eload(j + DR, (jm8 + DR) % DE)

        for k in range(DR):
            eload(k, k)
        eload_wait()
        eload_wait()
        gissue(0, 0)
        gissue(1, 1)

        for j in range(DR):
            do_slot(j, j, drain_scatter=(j >= 2), load_ahead=True,
                    issue_next=True)

        def steady(t, carry):
            j = DR + t * DE
            for u in range(DE):
                do_slot(j + u, (DR + u) % DE, drain_scatter=True,
                        load_ahead=True, issue_next=True)
            return carry
        lax.fori_loop(0, (NBLK_PER_TILE - 2 * DR) // DE, steady, 0)

        wmask = jnp.where(wid < 4, 1.0, 0.0).astype(jnp.float32)
        tail_masks = {RBPT: wmask, RBPT + 1: jnp.float32(0.0)}
        for j in range(NBLK_PER_TILE - DR, NBLK_PER_TILE):
            do_slot(j, j % DE, drain_scatter=True, load_ahead=False,
                    issue_next=(j < NBLK_PER_TILE - 2),
                    mask=tail_masks.get(j))
        for _ in range(2):
            swait()
        plsc.subcore_barrier()

        out_base = c * NP + s * ROWS_PER_TILE
        for k in range(ROWS_PER_TILE // ZR):
            pltpu.sync_copy(acc.at[pl.ds(s * ROWS_PER_TILE + k * ZR, ZR)],
                            out_hbm.at[pl.ds(out_base + k * ZR, ZR)])

    return sc_edge


def kernel(x, edge_index, batch, edge_attr, conv1_W, conv1_b, conv2_W,
           conv2_b, bn0_g, bn0_b, fc1_W, fc1_b, bn1_g, bn1_b, fc2_W, fc2_b,
           cls_W, cls_b):
    ei = edge_index.astype(jnp.int32)
    src = ei[0]
    dst = ei[1]
    ew = edge_attr.astype(jnp.float32)

    h0 = pl.pallas_call(
        _tc1_body,
        out_shape=jax.ShapeDtypeStruct((NP, NH), jnp.float32),
    )(x, conv1_W)

    s1p = _make_sc_edge(NH)(src, dst, ew, h0)

    h1, h2in = pl.pallas_call(
        _tc2_body,
        out_shape=(jax.ShapeDtypeStruct((N, NH), jnp.float32),
                   jax.ShapeDtypeStruct((NP, NO), jnp.float32)),
    )(s1p, conv1_b, conv2_W)

    s2p = _make_sc_edge(NO)(src, dst, ew, h2in)

    out = pl.pallas_call(
        _tc3_body,
        out_shape=jax.ShapeDtypeStruct((NG, NC), jnp.float32),
    )(h1, s2p, batch.astype(jnp.int32).reshape(N, 1), conv2_b,
      bn0_g, bn0_b, fc1_W, fc1_b, bn1_g, bn1_b, fc2_W, fc2_b, cls_W, cls_b)
    return out


# speedup vs baseline: 1.0627x; 1.0627x over previous
import functools

import jax
import jax.numpy as jnp
from jax import lax
from jax.experimental import pallas as pl
from jax.experimental.pallas import tpu as pltpu
from jax.experimental.pallas import tpu_sc as plsc

N = 10000
E = 320000
D = 128
NH = 64
NO = 16
NG = 128
NC = 10
MID = 256
LAST = 32
IN_MLP = 2 * NH + 2 * NO
EPS = 1e-5

NP = 10240
BLK = 128
NTILES = 32
NBLK_PER_TILE = 80
RBPT = 78
LEFT0 = NTILES * RBPT
DE = 8
DR = 4
ROWS_PER_TILE = NP // 16
ZR = 128
NEG = float("-inf")


def _dotT(a, b):
    return lax.dot_general(a, b, (((1,), (1,)), ((), ())),
                           preferred_element_type=jnp.float32)


def _tc1_body(x_ref, w_ref, o_ref):
    o_ref[0:N, :] = _dotT(x_ref[...], w_ref[...])
    o_ref[N:NP, :] = jnp.zeros((NP - N, NH), jnp.float32)


def _tc2_body(s1p_ref, b1_ref, w2_ref, h1_ref, h2_ref):
    s1 = s1p_ref[0:N, :] + s1p_ref[NP:NP + N, :]
    h1 = jnp.maximum(s1 + b1_ref[...], 0.0)
    h1_ref[...] = h1
    h2_ref[0:N, :] = _dotT(h1, w2_ref[...])
    h2_ref[N:NP, :] = jnp.zeros((NP - N, NO), jnp.float32)


def _bn_eval(x, g, b):
    return g * (x * (1.0 / (1.0 + EPS) ** 0.5)) + b


def _segmax(y, bb, C):
    s = 1
    while s < N:
        bs = jnp.concatenate(
            [jnp.full((s, 1), -1, jnp.int32), bb[:N - s]], axis=0)
        same = bb == bs
        ms = jnp.concatenate(
            [jnp.full((s, C), NEG, jnp.float32), y[:N - s]], axis=0)
        y = jnp.maximum(y, jnp.where(same, ms, NEG))
        s *= 2
    return y


def _pool(y, bb):
    C = y.shape[1]
    ymax = _segmax(y, bb, C)
    gidx = lax.broadcasted_iota(jnp.int32, (1, NG), 1)
    monehot = (bb == gidx).astype(jnp.float32)
    gsum = lax.dot_general(monehot, y, (((0,), (0,)), ((), ())),
                           preferred_element_type=jnp.float32)
    bnext = jnp.concatenate([bb[1:], jnp.full((1, 1), -1, jnp.int32)], axis=0)
    mend = monehot * (bnext != bb).astype(jnp.float32)
    gmax = lax.dot_general(mend, ymax, (((0,), (0,)), ((), ())),
                           preferred_element_type=jnp.float32)
    return gmax, gsum


def _tc3_body(h1_ref, s2p_ref, b_ref, b2_ref, bn0g_ref, bn0b_ref,
              fc1w_ref, fc1b_ref, bn1g_ref, bn1b_ref, fc2w_ref, fc2b_ref,
              clsw_ref, clsb_ref, o_ref):
    s2 = s2p_ref[0:N, :] + s2p_ref[NP:NP + N, :] + b2_ref[...]
    bb = b_ref[...]
    C = NH + NO
    y = jnp.concatenate([h1_ref[...], s2, jnp.ones((N, 1), jnp.float32)],
                        axis=1)
    gmax, gsum = _pool(y, bb)

    cnt = jnp.maximum(gsum[:, C:C + 1], 1.0)
    xf = jnp.concatenate([
        gmax[:, :NH], gsum[:, :NH] / cnt,
        gmax[:, NH:C], gsum[:, NH:C] / cnt,
    ], axis=1)

    xf = jnp.maximum(_dotT(_bn_eval(xf, bn0g_ref[...], bn0b_ref[...]),
                           fc1w_ref[...]) + fc1b_ref[...], 0.0)
    xf = jnp.maximum(_dotT(_bn_eval(xf, bn1g_ref[...], bn1b_ref[...]),
                           fc2w_ref[...]) + fc2b_ref[...], 0.0)
    logits = _dotT(xf, clsw_ref[...]) + clsb_ref[...]
    m = jnp.max(logits, axis=1, keepdims=True)
    lse = m + jnp.log(jnp.sum(jnp.exp(logits - m), axis=1, keepdims=True))
    o_ref[...] = logits - lse


def _make_sc_edge(F):
    mesh = plsc.VectorSubcoreMesh(core_axis_name="c", subcore_axis_name="s",
                                  num_cores=2, num_subcores=16)

    @functools.partial(
        pl.kernel,
        out_type=jax.ShapeDtypeStruct((2 * NP, F), jnp.float32),
        mesh=mesh,
        scratch_types=[
            pltpu.VMEM((DE, 2, BLK), jnp.int32),
            pltpu.VMEM((DE, BLK), jnp.float32),
            pltpu.VMEM((DR, BLK, F), jnp.float32),
            pltpu.VMEM((ZR, F), jnp.float32),
            pltpu.VMEM_SHARED((NP, F), jnp.float32),
            pltpu.VMEM_SHARED((NP, F), jnp.float32),
            pltpu.SemaphoreType.DMA,
            pltpu.SemaphoreType.DMA,
            pltpu.SemaphoreType.DMA,
        ],
        compiler_params=pltpu.CompilerParams(use_tc_tiling_on_sc=False),
    )
    def sc_edge(src_hbm, dst_hbm, ew_hbm, h_hbm, out_hbm, ebuf, wbuf, rows,
                zbuf, acc, hs, esem, gsem, ssem):
        c = lax.axis_index("c")
        s = lax.axis_index("s")
        wid = c * 16 + s

        def zb(i, carry):
            for q in range(F // 16):
                zbuf[i, pl.ds(q * 16, 16)] = jnp.zeros((16,), jnp.float32)
            return carry
        lax.fori_loop(0, ZR, zb, 0)
        for k in range(ROWS_PER_TILE // ZR):
            off = s * ROWS_PER_TILE + k * ZR
            pltpu.sync_copy(zbuf, acc.at[pl.ds(off, ZR)])
            pltpu.sync_copy(h_hbm.at[pl.ds(off, ZR)], hs.at[pl.ds(off, ZR)])
        plsc.subcore_barrier()

        blk0 = wid * RBPT

        def eload(j, slot):
            blk = jnp.where(j < RBPT, blk0 + j,
                            jnp.where(j == RBPT, LEFT0 + (wid % 4), 0))
            off = blk * BLK
            pltpu.async_copy(src_hbm.at[pl.ds(off, BLK)], ebuf.at[slot, 0],
                             esem)
            pltpu.async_copy(dst_hbm.at[pl.ds(off, BLK)], ebuf.at[slot, 1],
                             esem)
            pltpu.async_copy(ew_hbm.at[pl.ds(off, BLK)], wbuf.at[slot], esem)

        def eload_wait():
            pltpu.make_async_copy(src_hbm.at[pl.ds(0, BLK)], ebuf.at[0, 0],
                                  esem).wait()
            pltpu.make_async_copy(dst_hbm.at[pl.ds(0, BLK)], ebuf.at[0, 1],
                                  esem).wait()
            pltpu.make_async_copy(ew_hbm.at[pl.ds(0, BLK)], wbuf.at[0],
                                  esem).wait()

        def gissue(eslot, rslot):
            pltpu.async_copy(hs.at[ebuf.at[eslot, 0]], rows.at[rslot], gsem)

        def gwait():
            pltpu.make_async_copy(hs.at[ebuf.at[0, 0]], rows.at[0],
                                  gsem).wait()

        def sissue(eslot, rslot):
            pltpu.async_copy(rows.at[rslot], acc.at[ebuf.at[eslot, 1]], ssem,
                             add=True)

        def swait():
            pltpu.make_async_copy(rows.at[0], acc.at[ebuf.at[0, 1]],
                                  ssem).wait()

        def scale(eslot, rslot, mask=None):
            def sbody(e16, carry):
                ew16 = wbuf[eslot, pl.ds(e16 * 16, 16)]
                ew16 = jnp.where(ew16 != ew16, 0.0, ew16)
                if mask is not None:
                    ew16 = ew16 * mask
                base = e16 * 16
                for e in range(16):
                    sv = ew16[e]
                    for q in range(F // 16):
                        sl = pl.ds(q * 16, 16)
                        rows[rslot, base + e, sl] = rows[rslot, base + e, sl] * sv
                return carry
            lax.fori_loop(0, BLK // 16, sbody, 0)

        def do_slot(j, jm8, drain_scatter, load_ahead, issue_next,
                    mask=None):
            if drain_scatter:
                swait()
            if issue_next:
                eload_wait()
                gissue((jm8 + 1) % DE, (jm8 + 1) % DR)
            gwait()
            scale(jm8, jm8 % DR, mask)
            sissue(jm8, jm8 % DR)
            if load_ahead:
                eload(j + 4, (jm8 + 4) % DE)

        for k in range(4):
            eload(k, k)
        eload_wait()
        gissue(0, 0)

        for j in range(4):
            do_slot(j, j, drain_scatter=(j == 3), load_ahead=True,
                    issue_next=True)

        def steady(t, carry):
            j = 4 + t * DE
            for u in range(DE):
                do_slot(j + u, (4 + u) % DE, drain_scatter=True,
                        load_ahead=True, issue_next=True)
            return carry
        lax.fori_loop(0, (NBLK_PER_TILE - 8) // DE, steady, 0)

        wmask = jnp.where(wid < 4, 1.0, 0.0).astype(jnp.float32)
        tail_masks = {RBPT: wmask, RBPT + 1: jnp.float32(0.0)}
        for j in range(NBLK_PER_TILE - 4, NBLK_PER_TILE):
            do_slot(j, j % DE, drain_scatter=True, load_ahead=False,
                    issue_next=(j < NBLK_PER_TILE - 1),
                    mask=tail_masks.get(j))
        for _ in range(3):
            swait()
        plsc.subcore_barrier()

        out_base = c * NP + s * ROWS_PER_TILE
        for k in range(ROWS_PER_TILE // ZR):
            pltpu.sync_copy(acc.at[pl.ds(s * ROWS_PER_TILE + k * ZR, ZR)],
                            out_hbm.at[pl.ds(out_base + k * ZR, ZR)])

    return sc_edge


def kernel(x, edge_index, batch, edge_attr, conv1_W, conv1_b, conv2_W,
           conv2_b, bn0_g, bn0_b, fc1_W, fc1_b, bn1_g, bn1_b, fc2_W, fc2_b,
           cls_W, cls_b):
    ei = edge_index.astype(jnp.int32)
    src = ei[0]
    dst = ei[1]
    ew = edge_attr.astype(jnp.float32)

    h0 = pl.pallas_call(
        _tc1_body,
        out_shape=jax.ShapeDtypeStruct((NP, NH), jnp.float32),
    )(x, conv1_W)

    s1p = _make_sc_edge(NH)(src, dst, ew, h0)

    h1, h2in = pl.pallas_call(
        _tc2_body,
        out_shape=(jax.ShapeDtypeStruct((N, NH), jnp.float32),
                   jax.ShapeDtypeStruct((NP, NO), jnp.float32)),
    )(s1p, conv1_b, conv2_W)

    s2p = _make_sc_edge(NO)(src, dst, ew, h2in)

    out = pl.pallas_call(
        _tc3_body,
        out_shape=jax.ShapeDtypeStruct((NG, NC), jnp.float32),
    )(h1, s2p, batch.astype(jnp.int32).reshape(N, 1), conv2_b,
      bn0_g, bn0_b, fc1_W, fc1_b, bn1_g, bn1_b, fc2_W, fc2_b, cls_W, cls_b)
    return out


# speedup vs baseline: 1.1073x; 1.0420x over previous
import functools

import jax
import jax.numpy as jnp
from jax import lax
from jax.experimental import pallas as pl
from jax.experimental.pallas import tpu as pltpu
from jax.experimental.pallas import tpu_sc as plsc

N = 10000
E = 320000
D = 128
NH = 64
NO = 16
NG = 128
NC = 10
MID = 256
LAST = 32
IN_MLP = 2 * NH + 2 * NO
EPS = 1e-5

NP = 10240
BLK = 128
NTILES = 32
NBLK_PER_TILE = 80
RBPT = 78
LEFT0 = NTILES * RBPT
DE = 8
DR = 4
ROWS_PER_TILE = NP // 16
ZR = 128
NEG = float("-inf")


def _dotT(a, b):
    return lax.dot_general(a, b, (((1,), (1,)), ((), ())),
                           preferred_element_type=jnp.float32)


def _tc1_body(x_ref, w_ref, o_ref):
    o_ref[0:N, :] = _dotT(x_ref[...], w_ref[...])
    o_ref[N:NP, :] = jnp.zeros((NP - N, NH), jnp.float32)


def _tc2_body(s1p_ref, b1_ref, w2_ref, h1_ref, h2_ref):
    s1 = s1p_ref[0:N, :] + s1p_ref[NP:NP + N, :]
    h1 = jnp.maximum(s1 + b1_ref[...], 0.0)
    h1_ref[...] = h1
    h2_ref[0:N, :] = _dotT(h1, w2_ref[...])
    h2_ref[N:NP, :] = jnp.zeros((NP - N, NO), jnp.float32)


def _bn_eval(x, g, b):
    return g * (x * (1.0 / (1.0 + EPS) ** 0.5)) + b


def _segmax(y, bb, C):
    s = 1
    while s < N:
        bs = jnp.concatenate(
            [jnp.full((s, 1), -1, jnp.int32), bb[:N - s]], axis=0)
        same = bb == bs
        ms = jnp.concatenate(
            [jnp.full((s, C), NEG, jnp.float32), y[:N - s]], axis=0)
        y = jnp.maximum(y, jnp.where(same, ms, NEG))
        s *= 2
    return y


def _pool(y, bb):
    C = y.shape[1]
    ymax = _segmax(y, bb, C)
    gidx = lax.broadcasted_iota(jnp.int32, (1, NG), 1)
    monehot = (bb == gidx).astype(jnp.float32)
    gsum = lax.dot_general(monehot, y, (((0,), (0,)), ((), ())),
                           preferred_element_type=jnp.float32)
    bnext = jnp.concatenate([bb[1:], jnp.full((1, 1), -1, jnp.int32)], axis=0)
    mend = monehot * (bnext != bb).astype(jnp.float32)
    gmax = lax.dot_general(mend, ymax, (((0,), (0,)), ((), ())),
                           preferred_element_type=jnp.float32)
    return gmax, gsum


def _tc3_body(h1_ref, s2p_ref, b_ref, b2_ref, bn0g_ref, bn0b_ref,
              fc1w_ref, fc1b_ref, bn1g_ref, bn1b_ref, fc2w_ref, fc2b_ref,
              clsw_ref, clsb_ref, o_ref):
    s2 = s2p_ref[0:N, :] + s2p_ref[NP:NP + N, :] + b2_ref[...]
    bb = b_ref[...]
    C = NH + NO
    y = jnp.concatenate([h1_ref[...], s2, jnp.ones((N, 1), jnp.float32)],
                        axis=1)
    gmax, gsum = _pool(y, bb)

    cnt = jnp.maximum(gsum[:, C:C + 1], 1.0)
    xf = jnp.concatenate([
        gmax[:, :NH], gsum[:, :NH] / cnt,
        gmax[:, NH:C], gsum[:, NH:C] / cnt,
    ], axis=1)

    xf = jnp.maximum(_dotT(_bn_eval(xf, bn0g_ref[...], bn0b_ref[...]),
                           fc1w_ref[...]) + fc1b_ref[...], 0.0)
    xf = jnp.maximum(_dotT(_bn_eval(xf, bn1g_ref[...], bn1b_ref[...]),
                           fc2w_ref[...]) + fc2b_ref[...], 0.0)
    logits = _dotT(xf, clsw_ref[...]) + clsb_ref[...]
    m = jnp.max(logits, axis=1, keepdims=True)
    lse = m + jnp.log(jnp.sum(jnp.exp(logits - m), axis=1, keepdims=True))
    o_ref[...] = logits - lse


def _make_sc_edge(F):
    mesh = plsc.VectorSubcoreMesh(core_axis_name="c", subcore_axis_name="s",
                                  num_cores=2, num_subcores=16)

    @functools.partial(
        pl.kernel,
        out_type=jax.ShapeDtypeStruct((2 * NP, F), jnp.float32),
        mesh=mesh,
        scratch_types=[
            pltpu.VMEM((DE, 2, BLK), jnp.int32),
            pltpu.VMEM((DE, BLK), jnp.float32),
            pltpu.VMEM((DR, BLK, F), jnp.float32),
            pltpu.VMEM((ZR, F), jnp.float32),
            pltpu.VMEM_SHARED((NP, F), jnp.float32),
            pltpu.VMEM_SHARED((NP, F), jnp.float32),
            pltpu.SemaphoreType.DMA,
            pltpu.SemaphoreType.DMA,
            pltpu.SemaphoreType.DMA,
        ],
        compiler_params=pltpu.CompilerParams(use_tc_tiling_on_sc=False),
    )
    def sc_edge(ei_hbm, ew_hbm, h_hbm, out_hbm, ebuf, wbuf, rows,
                zbuf, acc, hs, esem, gsem, ssem):
        c = lax.axis_index("c")
        s = lax.axis_index("s")
        wid = c * 16 + s

        def zb(i, carry):
            for q in range(F // 16):
                zbuf[i, pl.ds(q * 16, 16)] = jnp.zeros((16,), jnp.float32)
            return carry
        lax.fori_loop(0, ZR, zb, 0)
        for k in range(ROWS_PER_TILE // ZR):
            off = s * ROWS_PER_TILE + k * ZR
            pltpu.sync_copy(zbuf, acc.at[pl.ds(off, ZR)])
            pltpu.sync_copy(h_hbm.at[pl.ds(off, ZR)], hs.at[pl.ds(off, ZR)])
        plsc.subcore_barrier()

        blk0 = wid * RBPT

        def eload(j, slot):
            blk = jnp.where(j < RBPT, blk0 + j,
                            jnp.where(j == RBPT, LEFT0 + (wid % 4), 0))
            off = blk * BLK
            pltpu.async_copy(ei_hbm.at[0, pl.ds(off, BLK)], ebuf.at[slot, 0],
                             esem)
            pltpu.async_copy(ei_hbm.at[1, pl.ds(off, BLK)], ebuf.at[slot, 1],
                             esem)
            pltpu.async_copy(ew_hbm.at[pl.ds(off, BLK)], wbuf.at[slot], esem)

        def eload_wait():
            pltpu.make_async_copy(ei_hbm.at[0, pl.ds(0, BLK)], ebuf.at[0, 0],
                                  esem).wait()
            pltpu.make_async_copy(ei_hbm.at[1, pl.ds(0, BLK)], ebuf.at[0, 1],
                                  esem).wait()
            pltpu.make_async_copy(ew_hbm.at[pl.ds(0, BLK)], wbuf.at[0],
                                  esem).wait()

        def gissue(eslot, rslot):
            pltpu.async_copy(hs.at[ebuf.at[eslot, 0]], rows.at[rslot], gsem)

        def gwait():
            pltpu.make_async_copy(hs.at[ebuf.at[0, 0]], rows.at[0],
                                  gsem).wait()

        def sissue(eslot, rslot):
            pltpu.async_copy(rows.at[rslot], acc.at[ebuf.at[eslot, 1]], ssem,
                             add=True)

        def swait():
            pltpu.make_async_copy(rows.at[0], acc.at[ebuf.at[0, 1]],
                                  ssem).wait()

        def scale(eslot, rslot, mask=None):
            def sbody(e16, carry):
                ew16 = wbuf[eslot, pl.ds(e16 * 16, 16)]
                ew16 = jnp.where(ew16 != ew16, 0.0, ew16)
                if mask is not None:
                    ew16 = ew16 * mask
                base = e16 * 16
                for e in range(16):
                    sv = ew16[e]
                    for q in range(F // 16):
                        sl = pl.ds(q * 16, 16)
                        rows[rslot, base + e, sl] = rows[rslot, base + e, sl] * sv
                return carry
            lax.fori_loop(0, BLK // 16, sbody, 0)

        def do_slot(j, jm8, drain_scatter, load_ahead, issue_next,
                    mask=None):
            if drain_scatter:
                swait()
            if issue_next:
                eload_wait()
                gissue((jm8 + 1) % DE, (jm8 + 1) % DR)
            gwait()
            scale(jm8, jm8 % DR, mask)
            sissue(jm8, jm8 % DR)
            if load_ahead:
                eload(j + 4, (jm8 + 4) % DE)

        for k in range(4):
            eload(k, k)
        eload_wait()
        gissue(0, 0)

        for j in range(4):
            do_slot(j, j, drain_scatter=(j == 3), load_ahead=True,
                    issue_next=True)

        def steady(t, carry):
            j = 4 + t * DE
            for u in range(DE):
                do_slot(j + u, (4 + u) % DE, drain_scatter=True,
                        load_ahead=True, issue_next=True)
            return carry
        lax.fori_loop(0, (NBLK_PER_TILE - 8) // DE, steady, 0)

        wmask = jnp.where(wid < 4, 1.0, 0.0).astype(jnp.float32)
        tail_masks = {RBPT: wmask, RBPT + 1: jnp.float32(0.0)}
        for j in range(NBLK_PER_TILE - 4, NBLK_PER_TILE):
            do_slot(j, j % DE, drain_scatter=True, load_ahead=False,
                    issue_next=(j < NBLK_PER_TILE - 1),
                    mask=tail_masks.get(j))
        for _ in range(3):
            swait()
        plsc.subcore_barrier()

        out_base = c * NP + s * ROWS_PER_TILE
        for k in range(ROWS_PER_TILE // ZR):
            pltpu.sync_copy(acc.at[pl.ds(s * ROWS_PER_TILE + k * ZR, ZR)],
                            out_hbm.at[pl.ds(out_base + k * ZR, ZR)])

    return sc_edge


def kernel(x, edge_index, batch, edge_attr, conv1_W, conv1_b, conv2_W,
           conv2_b, bn0_g, bn0_b, fc1_W, fc1_b, bn1_g, bn1_b, fc2_W, fc2_b,
           cls_W, cls_b):
    ei = edge_index.astype(jnp.int32)
    ew = edge_attr.astype(jnp.float32)

    h0 = pl.pallas_call(
        _tc1_body,
        out_shape=jax.ShapeDtypeStruct((NP, NH), jnp.float32),
    )(x, conv1_W)

    s1p = _make_sc_edge(NH)(ei, ew, h0)

    h1, h2in = pl.pallas_call(
        _tc2_body,
        out_shape=(jax.ShapeDtypeStruct((N, NH), jnp.float32),
                   jax.ShapeDtypeStruct((NP, NO), jnp.float32)),
    )(s1p, conv1_b, conv2_W)

    s2p = _make_sc_edge(NO)(ei, ew, h2in)

    out = pl.pallas_call(
        _tc3_body,
        out_shape=jax.ShapeDtypeStruct((NG, NC), jnp.float32),
    )(h1, s2p, batch.astype(jnp.int32).reshape(N, 1), conv2_b,
      bn0_g, bn0_b, fc1_W, fc1_b, bn1_g, bn1_b, fc2_W, fc2_b, cls_W, cls_b)
    return out
